# Initial kernel scaffold; baseline (speedup 1.0000x reference)
import jax
import jax.numpy as jnp
from jax.experimental import pallas as pl


def kernel(x, edge_index, W_emb, b_emb, Wq, bq, Wk, bk, Wv, bv, Wo, bo, W1, b1, W2, b2, Wr0, br0, Wr1, br1, Wr2, br2):
    raise NotImplementedError("write your pallas kernel here")


# speedup vs baseline: 10.3112x; 10.3112x over previous
import functools
import math

import jax
import jax.numpy as jnp
from jax import lax
from jax.experimental import pallas as pl
from jax.experimental.pallas import tpu as pltpu

N_NODES = 10000
N_EDGES = 160000
IN_DIM = 9
HID = 80
NH = 8
DH = 10
DHP = 16
HIDP = NH * DHP
ACC_W = 96
N_LAYERS = 10
ROW_BLK = 1000


def _tc_call(body, out_shapes, in_specs, out_specs, grid):
    return pl.pallas_call(
        body,
        grid=grid,
        in_specs=in_specs,
        out_specs=out_specs,
        out_shape=out_shapes,
    )


def _full(spec_shape):
    return pl.BlockSpec(spec_shape, lambda i: tuple(0 for _ in spec_shape))


def _rows(d):
    return pl.BlockSpec((ROW_BLK, d), lambda i: (i, 0))


def _emb_body(x_ref, w_ref, b_ref, o_ref):
    o_ref[...] = jnp.dot(x_ref[...], w_ref[...],
                         preferred_element_type=jnp.float32, precision=jax.lax.Precision.HIGHEST) + b_ref[...]


def _embed(x, W_emb, b_emb):
    return _tc_call(
        _emb_body,
        jax.ShapeDtypeStruct((N_NODES, HID), jnp.float32),
        [_rows(IN_DIM), _full((IN_DIM, HID)), _full((1, HID))],
        _rows(HID),
        grid=(N_NODES // ROW_BLK,),
    )(x, W_emb, b_emb.reshape(1, HID))


def _qkv_body(h_ref, wq_ref, bq_ref, wk_ref, bk_ref, wv_ref, bv_ref,
              q_ref, k_ref, v_ref):
    h = h_ref[...]
    q_ref[...] = jnp.dot(h, wq_ref[...], preferred_element_type=jnp.float32, precision=jax.lax.Precision.HIGHEST) + bq_ref[...]
    k_ref[...] = jnp.dot(h, wk_ref[...], preferred_element_type=jnp.float32, precision=jax.lax.Precision.HIGHEST) + bk_ref[...]
    v_ref[...] = jnp.dot(h, wv_ref[...], preferred_element_type=jnp.float32, precision=jax.lax.Precision.HIGHEST) + bv_ref[...]


def _qkv(h, Wqp, bqp, Wkp, bkp, Wv, bv):
    return _tc_call(
        _qkv_body,
        (jax.ShapeDtypeStruct((N_NODES, HIDP), jnp.float32),
         jax.ShapeDtypeStruct((N_NODES, HIDP), jnp.float32),
         jax.ShapeDtypeStruct((N_NODES, HID), jnp.float32)),
        [_rows(HID),
         _full((HID, HIDP)), _full((1, HIDP)),
         _full((HID, HIDP)), _full((1, HIDP)),
         _full((HID, HID)), _full((1, HID))],
        (_rows(HIDP), _rows(HIDP), _rows(HID)),
        grid=(N_NODES // ROW_BLK,),
    )(h, Wqp, bqp.reshape(1, HIDP), Wkp, bkp.reshape(1, HIDP), Wv, bv.reshape(1, HID))


def _post_body(h_ref, a0_ref, a1_ref, wo_ref, bo_ref, w1_ref, b1_ref,
               w2_ref, b2_ref, o_ref, attn_ref):
    acc = a0_ref[...] + a1_ref[...]
    for hh in range(NH):
        z = acc[:, HID + hh:HID + hh + 1] + 1e-6
        attn_ref[:, hh * DH:(hh + 1) * DH] = acc[:, hh * DH:(hh + 1) * DH] / z
    h1 = h_ref[...] + jnp.dot(attn_ref[...], wo_ref[...],
                              preferred_element_type=jnp.float32, precision=jax.lax.Precision.HIGHEST) + bo_ref[...]
    hf = jnp.maximum(jnp.dot(h1, w1_ref[...], preferred_element_type=jnp.float32, precision=jax.lax.Precision.HIGHEST)
                     + b1_ref[...], 0.0)
    hf = jnp.dot(hf, w2_ref[...], preferred_element_type=jnp.float32, precision=jax.lax.Precision.HIGHEST) + b2_ref[...]
    o_ref[...] = h1 + hf


def _post(h, acc0, acc1, Wo, bo, W1, b1, W2, b2):
    return pl.pallas_call(
        _post_body,
        grid=(N_NODES // ROW_BLK,),
        in_specs=[_rows(HID), _rows(ACC_W), _rows(ACC_W),
                  _full((HID, HID)), _full((1, HID)),
                  _full((HID, 2 * HID)), _full((1, 2 * HID)),
                  _full((2 * HID, HID)), _full((1, HID))],
        out_specs=_rows(HID),
        out_shape=jax.ShapeDtypeStruct((N_NODES, HID), jnp.float32),
        scratch_shapes=[pltpu.VMEM((ROW_BLK, HID), jnp.float32)],
    )(h, acc0, acc1, Wo, bo.reshape(1, HID), W1, b1.reshape(1, 2 * HID),
      W2, b2.reshape(1, HID))


def _read_body(h_ref, w0_ref, b0_ref, w1_ref, b1_ref, w2_ref, b2_ref, o_ref):
    t = jnp.maximum(jnp.dot(h_ref[...], w0_ref[...],
                            preferred_element_type=jnp.float32, precision=jax.lax.Precision.HIGHEST) + b0_ref[...], 0.0)
    t = jnp.maximum(jnp.dot(t, w1_ref[...],
                            preferred_element_type=jnp.float32, precision=jax.lax.Precision.HIGHEST) + b1_ref[...], 0.0)
    o_ref[...] = jnp.dot(t, w2_ref[...],
                         preferred_element_type=jnp.float32, precision=jax.lax.Precision.HIGHEST) + b2_ref[...]


def _readout(h, Wr0, br0, Wr1, br1, Wr2, br2):
    return _tc_call(
        _read_body,
        jax.ShapeDtypeStruct((N_NODES, 4), jnp.float32),
        [_rows(HID),
         _full((HID, HID // 2)), _full((1, HID // 2)),
         _full((HID // 2, HID // 4)), _full((1, HID // 4)),
         _full((HID // 4, 4)), _full((1, 4))],
        _rows(4),
        grid=(N_NODES // ROW_BLK,),
    )(h, Wr0, br0.reshape(1, -1), Wr1, br1.reshape(1, -1), Wr2, br2.reshape(1, -1))


def _pad_headmajor(W, b, scale):
    Wr = (W * scale).reshape(W.shape[0], NH, DH)
    Wp = jnp.pad(Wr, ((0, 0), (0, 0), (0, DHP - DH)))
    bp = jnp.pad((b * scale).reshape(NH, DH), ((0, 0), (0, DHP - DH)))
    return Wp.reshape(W.shape[0], NH * DHP), bp.reshape(NH * DHP)


def _edge_attention(Qp, Kp, V, src, dst):
    qe = Qp[dst].reshape(N_EDGES, NH, DHP)
    ke = Kp[src].reshape(N_EDGES, NH, DHP)
    score = jnp.exp(jnp.clip(jnp.sum(qe * ke, axis=-1), -5.0, 5.0))
    sv = V[src].reshape(N_EDGES, NH, DH) * score[:, :, None]
    rows = jnp.concatenate([sv.reshape(N_EDGES, HID), score,
                            jnp.zeros((N_EDGES, ACC_W - HID - NH), jnp.float32)], axis=1)
    acc = jax.ops.segment_sum(rows, dst, num_segments=N_NODES)
    return acc, jnp.zeros_like(acc)


def kernel(x, edge_index, W_emb, b_emb, Wq, bq, Wk, bk, Wv, bv, Wo, bo,
           W1, b1, W2, b2, Wr0, br0, Wr1, br1, Wr2, br2):
    src = edge_index[0]
    dst = edge_index[1]
    inv = 1.0 / math.sqrt(float(DH))

    h = _embed(x, W_emb, b_emb)
    for l in range(N_LAYERS):
        Wqp, bqp = _pad_headmajor(Wq[l], bq[l], inv)
        Wkp, bkp = _pad_headmajor(Wk[l], bk[l], 1.0)
        Qp, Kp, V = _qkv(h, Wqp, bqp, Wkp, bkp, Wv[l], bv[l])
        acc0, acc1 = _edge_attention(Qp, Kp, V, src, dst)
        h = _post(h, acc0, acc1, Wo[l], bo[l], W1[l], b1[l], W2[l], b2[l])
    return _readout(h, Wr0, br0, Wr1, br1, Wr2, br2)


# speedup vs baseline: 20.2590x; 1.9648x over previous
import functools
import math

import jax
import jax.numpy as jnp
from jax import lax
from jax.experimental import pallas as pl
from jax.experimental.pallas import tpu as pltpu
from jax.experimental.pallas import tpu_sc as plsc

N_NODES = 10000
N_EDGES = 160000
IN_DIM = 9
HID = 80
NH = 8
DH = 10
DHP = 16
HIDP = NH * DHP
N_LAYERS = 10
ROW_BLK = 1000

_C = 32
_RPS = 624
_ZCHUNK = 208
_REM_BASE = 16 * _RPS
_PREC = jax.lax.Precision.DEFAULT


def _full(spec_shape):
    return pl.BlockSpec(spec_shape, lambda i: tuple(0 for _ in spec_shape))


def _rows(d):
    return pl.BlockSpec((ROW_BLK, d), lambda i: (i, 0))


def _emb_body(x_ref, w_ref, b_ref, o_ref):
    o_ref[...] = jnp.dot(x_ref[...], w_ref[...],
                         preferred_element_type=jnp.float32, precision=_PREC) + b_ref[...]


def _embed(x, W_emb, b_emb):
    return pl.pallas_call(
        _emb_body,
        grid=(N_NODES // ROW_BLK,),
        in_specs=[_rows(IN_DIM), _full((IN_DIM, HID)), _full((1, HID))],
        out_specs=_rows(HID),
        out_shape=jax.ShapeDtypeStruct((N_NODES, HID), jnp.float32),
    )(x, W_emb, b_emb.reshape(1, HID))


def _qkv_body(h_ref, wq_ref, bq_ref, wkv_ref, bkv_ref, q_ref, kv_ref):
    h = h_ref[...]
    q_ref[...] = jnp.dot(h, wq_ref[...], preferred_element_type=jnp.float32,
                         precision=_PREC) + bq_ref[...]
    kv_ref[...] = jnp.dot(h, wkv_ref[...], preferred_element_type=jnp.float32,
                          precision=_PREC) + bkv_ref[...]


def _qkv(h, Wqp, bqp, Wkv, bkv):
    return pl.pallas_call(
        _qkv_body,
        grid=(N_NODES // ROW_BLK,),
        in_specs=[_rows(HID),
                  _full((HID, HIDP)), _full((1, HIDP)),
                  _full((HID, 2 * HIDP)), _full((1, 2 * HIDP))],
        out_specs=(_rows(HIDP), _rows(2 * HIDP)),
        out_shape=(jax.ShapeDtypeStruct((N_NODES, HIDP), jnp.float32),
                   jax.ShapeDtypeStruct((N_NODES, 2 * HIDP), jnp.float32)),
    )(h, Wqp, bqp.reshape(1, HIDP), Wkv, bkv.reshape(1, 2 * HIDP))


def _post_body(h_ref, a0_ref, a1_ref, wo_ref, bo_ref, w1_ref, b1_ref,
               w2_ref, b2_ref, o_ref, attn_ref):
    acc = a0_ref[0] + a1_ref[0]
    for hh in range(NH):
        z = acc[:, DHP * hh + DH:DHP * hh + DH + 1] + 1e-6
        attn_ref[:, hh * DH:(hh + 1) * DH] = acc[:, DHP * hh:DHP * hh + DH] / z
    h1 = h_ref[...] + jnp.dot(attn_ref[...], wo_ref[...],
                              preferred_element_type=jnp.float32,
                              precision=_PREC) + bo_ref[...]
    hf = jnp.maximum(jnp.dot(h1, w1_ref[...], preferred_element_type=jnp.float32,
                             precision=_PREC) + b1_ref[...], 0.0)
    hf = jnp.dot(hf, w2_ref[...], preferred_element_type=jnp.float32,
                 precision=_PREC) + b2_ref[...]
    o_ref[...] = h1 + hf


def _post(h, acc, Wo, bo, W1, b1, W2, b2):
    return pl.pallas_call(
        _post_body,
        grid=(N_NODES // ROW_BLK,),
        in_specs=[_rows(HID),
                  pl.BlockSpec((1, ROW_BLK, HIDP), lambda i: (0, i, 0)),
                  pl.BlockSpec((1, ROW_BLK, HIDP), lambda i: (1, i, 0)),
                  _full((HID, HID)), _full((1, HID)),
                  _full((HID, 2 * HID)), _full((1, 2 * HID)),
                  _full((2 * HID, HID)), _full((1, HID))],
        out_specs=_rows(HID),
        out_shape=jax.ShapeDtypeStruct((N_NODES, HID), jnp.float32),
        scratch_shapes=[pltpu.VMEM((ROW_BLK, HID), jnp.float32)],
    )(h, acc, acc, Wo, bo.reshape(1, HID), W1, b1.reshape(1, 2 * HID),
      W2, b2.reshape(1, HID))


def _read_body(h_ref, w0_ref, b0_ref, w1_ref, b1_ref, w2_ref, b2_ref, o_ref):
    t = jnp.maximum(jnp.dot(h_ref[...], w0_ref[...],
                            preferred_element_type=jnp.float32,
                            precision=_PREC) + b0_ref[...], 0.0)
    t = jnp.maximum(jnp.dot(t, w1_ref[...], preferred_element_type=jnp.float32,
                            precision=_PREC) + b1_ref[...], 0.0)
    o_ref[...] = jnp.dot(t, w2_ref[...], preferred_element_type=jnp.float32,
                         precision=_PREC) + b2_ref[...]


def _readout(h, Wr0, br0, Wr1, br1, Wr2, br2):
    return pl.pallas_call(
        _read_body,
        grid=(N_NODES // ROW_BLK,),
        in_specs=[_rows(HID),
                  _full((HID, HID // 2)), _full((1, HID // 2)),
                  _full((HID // 2, HID // 4)), _full((1, HID // 4)),
                  _full((HID // 4, 4)), _full((1, 4))],
        out_specs=_rows(4),
        out_shape=jax.ShapeDtypeStruct((N_NODES, 4), jnp.float32),
    )(h, Wr0, br0.reshape(1, -1), Wr1, br1.reshape(1, -1), Wr2, br2.reshape(1, -1))


def _pad_headmajor(W, b, scale, lane10_bias=0.0):
    Wr = (W * scale).reshape(W.shape[0], NH, DH)
    Wp = jnp.pad(Wr, ((0, 0), (0, 0), (0, DHP - DH)))
    bp = jnp.pad((b * scale).reshape(NH, DH), ((0, 0), (0, DHP - DH)))
    if lane10_bias:
        bp = bp.at[:, DH].set(lane10_bias)
    return Wp.reshape(W.shape[0], NH * DHP), bp.reshape(NH * DHP)


_ACC_ROWS = N_NODES + 16
_TRASH = N_NODES


def _sc_edge_body(qp_hbm, kv_hbm, src_hbm, dst_hbm, params_hbm, out_hbm,
                  src_v, dst_v, sdst_v, params_v, kvrows, qrows, orows, zbuf, acc,
                  sem0, sem1):
    c = lax.axis_index("c")
    s = lax.axis_index("s")
    w = s * 2 + c

    zero16 = jnp.zeros((16,), jnp.float32)

    def _zb(i, carry):
        zbuf[i // (HIDP // 16), pl.ds((i % (HIDP // 16)) * 16, 16)] = zero16
        return carry

    lax.fori_loop(0, _ZCHUNK * (HIDP // 16), _zb, 0)
    for j in range(_RPS // _ZCHUNK):
        pltpu.sync_copy(zbuf, acc.at[pl.ds(s * _RPS + j * _ZCHUNK, _ZCHUNK)])

    @pl.when(s == 0)
    def _zrem():
        pltpu.sync_copy(zbuf.at[pl.ds(0, 16)], acc.at[pl.ds(_REM_BASE, 16)])

    plsc.subcore_barrier()

    li = lax.iota(jnp.int32, 16)
    x8, x4, x2, x1 = li ^ 8, li ^ 4, li ^ 2, li ^ 1
    m8 = (li & 8) == 0
    m4 = (li & 4) == 0
    m2 = (li & 2) == 0
    hsp = []
    for h in range(NH):
        hv = jnp.full((16,), h, jnp.int32)
        hsp.append(((hv >> 2) & 1) * 2 + ((hv >> 1) & 1) * 4 + (hv & 1) * 8)

    _gdn = lax.GatherDimensionNumbers(offset_dims=(), collapsed_slice_dims=(0,),
                                      start_index_map=(0,))

    def vperm(v, idx):
        return lax.gather(v, idx.reshape(16, 1), _gdn, (1,),
                          mode=lax.GatherScatterMode.PROMISE_IN_BOUNDS)

    def fold(v, xk):
        return v + vperm(v, xk)

    def _edge(e, carry):
        pr = [kvrows[e, pl.ds(16 * h, 16)] * qrows[e, pl.ds(16 * h, 16)]
              for h in range(NH)]
        m1 = [jnp.where(m8, fold(pr[2 * j], x8), fold(pr[2 * j + 1], x8))
              for j in range(4)]
        m2v0 = jnp.where(m4, fold(m1[0], x4), fold(m1[1], x4))
        m2v1 = jnp.where(m4, fold(m1[2], x4), fold(m1[3], x4))
        m3 = jnp.where(m2, fold(m2v0, x2), fold(m2v1, x2))
        f = fold(m3, x1) * (1.0 / math.sqrt(float(DH)))
        sc = jnp.exp(jnp.minimum(jnp.maximum(f, -5.0), 5.0))
        for h in range(NH):
            orows[e, pl.ds(16 * h, 16)] = (kvrows[e, pl.ds(HIDP + 16 * h, 16)]
                                           * vperm(sc, hsp[h]))
        return carry

    pltpu.sync_copy(params_hbm, params_v)
    pvec = params_v[pl.ds(4 * w, 16)]
    b0 = pvec[0]
    b1 = pvec[1]
    clo = pvec[2]
    nch = pvec[3]

    def _chunk(kk, carry):
        base = pl.multiple_of((clo + kk) * _C, 8)
        pltpu.sync_copy(src_hbm.at[pl.ds(base, _C)], src_v)
        pltpu.sync_copy(dst_hbm.at[pl.ds(base, _C)], dst_v)
        pltpu.async_copy(kv_hbm.at[src_v], kvrows, sem0).wait()
        pltpu.async_copy(qp_hbm.at[dst_v], qrows, sem1).wait()
        for j in range(_C // 16):
            ev = base + 16 * j + li
            dv = dst_v[pl.ds(16 * j, 16)]
            ok = jnp.logical_and(ev >= b0, ev < b1)
            sdst_v[pl.ds(16 * j, 16)] = jnp.where(ok, dv, _TRASH)
        lax.fori_loop(0, _C, _edge, 0)
        pltpu.sync_copy(orows, acc.at[sdst_v], add=True)
        return carry

    lax.fori_loop(0, nch, _chunk, 0)
    plsc.subcore_barrier()

    for j in range(_RPS // _ZCHUNK):
        r0 = s * _RPS + j * _ZCHUNK
        pltpu.sync_copy(acc.at[pl.ds(r0, _ZCHUNK)], out_hbm.at[c, pl.ds(r0, _ZCHUNK)])

    @pl.when(s == 0)
    def _wrem():
        pltpu.sync_copy(acc.at[pl.ds(_REM_BASE, 16)],
                        out_hbm.at[c, pl.ds(_REM_BASE, 16)])


def _edge_attention(Qp, KV, src_s, dst_s, params):
    mesh = plsc.VectorSubcoreMesh(core_axis_name="c", subcore_axis_name="s")
    f = pl.kernel(
        _sc_edge_body,
        out_type=jax.ShapeDtypeStruct((2, N_NODES, HIDP), jnp.float32),
        mesh=mesh,
        scratch_types=[
            pltpu.VMEM((_C,), jnp.int32),
            pltpu.VMEM((_C,), jnp.int32),
            pltpu.VMEM((_C,), jnp.int32),
            pltpu.VMEM((160,), jnp.int32),
            pltpu.VMEM((_C, 2 * HIDP), jnp.float32),
            pltpu.VMEM((_C, HIDP), jnp.float32),
            pltpu.VMEM((_C, HIDP), jnp.float32),
            pltpu.VMEM((_ZCHUNK, HIDP), jnp.float32),
            pltpu.VMEM_SHARED((_ACC_ROWS, HIDP), jnp.float32),
            pltpu.SemaphoreType.DMA,
            pltpu.SemaphoreType.DMA,
        ],
    )
    return f(Qp, KV, src_s, dst_s, params)


def _edge_plan(src, dst):
    order = jnp.argsort(dst, stable=True)
    src_s = src[order]
    dst_s = dst[order]
    e_t = jnp.arange(32, dtype=jnp.int32) * (N_EDGES // 32)
    n_t = dst_s[e_t]
    b = jnp.searchsorted(dst_s, n_t, side="left").astype(jnp.int32)
    b_next = jnp.concatenate([b[1:], jnp.array([N_EDGES], jnp.int32)])
    clo = b // _C
    nch = jnp.where(b_next > b, (b_next + _C - 1) // _C - clo, 0)
    params = jnp.stack([b, b_next, clo, nch], axis=1).reshape(128)
    params = jnp.concatenate([params, jnp.zeros((32,), jnp.int32)])
    return src_s, dst_s, params


def kernel(x, edge_index, W_emb, b_emb, Wq, bq, Wk, bk, Wv, bv, Wo, bo,
           W1, b1, W2, b2, Wr0, br0, Wr1, br1, Wr2, br2):
    src = edge_index[0]
    dst = edge_index[1]
    src_s, dst_s, params = _edge_plan(src, dst)

    h = _embed(x, W_emb, b_emb)
    for l in range(N_LAYERS):
        Wqp, bqp = _pad_headmajor(Wq[l], bq[l], 1.0)
        Wkp, bkp = _pad_headmajor(Wk[l], bk[l], 1.0)
        Wvp, bvp = _pad_headmajor(Wv[l], bv[l], 1.0, lane10_bias=1.0)
        Wkv = jnp.concatenate([Wkp, Wvp], axis=1)
        bkv = jnp.concatenate([bkp, bvp])
        Qp, KV = _qkv(h, Wqp, bqp, Wkv, bkv)
        acc = _edge_attention(Qp, KV, src_s, dst_s, params)
        h = _post(h, acc, Wo[l], bo[l], W1[l], b1[l], W2[l], b2[l])
    return _readout(h, Wr0, br0, Wr1, br1, Wr2, br2)
